# baseline (speedup 1.0000x reference)
import functools

import jax
import jax.numpy as jnp
from jax import lax
from jax.experimental import pallas as pl
from jax.experimental.pallas import tpu as pltpu

EPS = 1e-5
VMEM_LIMIT = 64 * 1024 * 1024


def _fused_kernel(nh_ref, eh_ref, idx_ref, mi_ref, mij_ref,
                  nnw_ref, nnb_ref, enw_ref, enbc_ref,
                  wmi_ref, wmj_ref, wme_ref, mb1_ref, mw2_ref, mb2_ref,
                  u1n_ref, u1m_ref, ub1_ref, uw2_ref, ub2_ref,
                  wei_ref, wej_ref, wee_ref, eb1_ref, ew2_ref, eb2_ref,
                  no_ref, eo_ref,
                  en_scr,
                  *, K, C):
    N, Dn = nh_ref.shape
    TE, De = eh_ref.shape
    CH = TE // C
    Tc = CH // K
    bf16 = jnp.bfloat16
    f32 = jnp.float32

    enw = enw_ref[...]
    enbc = enbc_ref[...]
    wme = (enw * wme_ref[...]).astype(bf16)
    wee = (enw * wee_ref[...]).astype(bf16)
    mb1 = mb1_ref[...] + jnp.sum(enbc * wme_ref[...], axis=0, keepdims=True)
    eb1 = eb1_ref[...] + jnp.sum(enbc * wee_ref[...], axis=0, keepdims=True)
    wmi = wmi_ref[...].astype(bf16)
    wmj = wmj_ref[...].astype(bf16)
    wei = wei_ref[...].astype(bf16)
    wej = wej_ref[...].astype(bf16)
    u1n = u1n_ref[...].astype(bf16)
    u1m = u1m_ref[...].astype(bf16)
    mw2 = mw2_ref[...].astype(bf16)
    uw2 = uw2_ref[...].astype(bf16)
    ew2 = ew2_ref[...].astype(bf16)

    s_sel = (lax.broadcasted_iota(jnp.int32, (Tc, CH), 1) // K
             == lax.broadcasted_iota(jnp.int32, (Tc, CH), 0)).astype(bf16)

    nh = nh_ref[...]
    mi = mi_ref[...]
    mu = jnp.mean(nh, axis=-1, keepdims=True)
    var = jnp.mean(nh * nh, axis=-1, keepdims=True) - mu * mu
    nhn = ((nh - mu) * lax.rsqrt(var + EPS) * nnw_ref[...] + nnb_ref[...]) * mi
    nhn_b = nhn.astype(bf16)

    pre_i = jnp.dot(nhn_b, wmi, preferred_element_type=f32) + mb1
    tblw_m = jnp.dot(nhn_b, wmj, preferred_element_type=f32).astype(bf16)

    iota = lax.broadcasted_iota(jnp.int32, (CH, N), 1)

    hsums, msums = [], []
    for c in range(C):
        sl = pl.ds(c * CH, CH)
        e = eh_ref[sl, :]
        mu_e = jnp.mean(e, axis=-1, keepdims=True)
        var_e = jnp.mean(e * e, axis=-1, keepdims=True) - mu_e * mu_e
        mij = mij_ref[sl, :]
        en_b = ((e - mu_e) * lax.rsqrt(var_e + EPS)).astype(bf16)
        en_scr[sl, :] = en_b

        oh = (idx_ref[sl, :] == iota).astype(bf16)

        zj = jnp.dot(oh, tblw_m, preferred_element_type=f32)
        ze = jnp.dot(en_b, wme, preferred_element_type=f32)
        z = (zj + ze).reshape(Tc, K, -1) + pre_i[c * Tc:(c + 1) * Tc][:, None, :]
        h = (jnp.maximum(z, 0.0).reshape(CH, -1) * mij).astype(bf16)
        mij_b = mij.astype(bf16)
        hsums.append(jnp.dot(s_sel, h, preferred_element_type=f32))
        msums.append(jnp.dot(s_sel, mij_b, preferred_element_type=f32))

    hsum = jnp.concatenate(hsums, axis=0)
    msum = jnp.concatenate(msums, axis=0)
    msg = (jnp.dot(hsum.astype(bf16), mw2, preferred_element_type=f32)
           + mb2_ref[...] * msum)

    u = jnp.maximum(
        jnp.dot(nhn_b, u1n, preferred_element_type=f32)
        + jnp.dot(msg.astype(bf16), u1m, preferred_element_type=f32)
        + ub1_ref[...], 0.0)
    upd = jnp.dot(u.astype(bf16), uw2, preferred_element_type=f32) + ub2_ref[...]
    nout = (nh + upd) * mi
    no_ref[...] = nout

    nout_b = nout.astype(bf16)
    pre_e = jnp.dot(nout_b, wei, preferred_element_type=f32) + eb1
    tblw_e = jnp.dot(nout_b, wej, preferred_element_type=f32).astype(bf16)

    for c in range(C):
        sl = pl.ds(c * CH, CH)
        oh = (idx_ref[sl, :] == iota).astype(bf16)
        en_b = en_scr[sl, :]
        zj = jnp.dot(oh, tblw_e, preferred_element_type=f32)
        ze = jnp.dot(en_b, wee, preferred_element_type=f32)
        z = (zj + ze).reshape(Tc, K, -1) + pre_e[c * Tc:(c + 1) * Tc][:, None, :]
        h = jnp.maximum(z, 0.0).reshape(CH, -1)
        upd_e = jnp.dot(h.astype(bf16), ew2, preferred_element_type=f32) + eb2_ref[...]
        eo_ref[sl, :] = (eh_ref[sl, :] + upd_e) * mij_ref[sl, :]


def kernel(node_h, edge_h, edge_idx, mask_i, mask_ij,
           node_norm_w, node_norm_b, edge_norm_w, edge_norm_b,
           msg_W1, msg_b1, msg_W2, msg_b2,
           upd_W1, upd_b1, upd_W2, upd_b2,
           edge_W1, edge_b1, edge_W2, edge_b2):
    B, N, Dn = node_h.shape
    K = edge_idx.shape[-1]
    De = edge_h.shape[-1]
    Hm = msg_W2.shape[0]
    Hu = upd_W2.shape[0]
    He = edge_W2.shape[0]
    TE = N * K
    C = 8
    f32 = jnp.float32

    eh2 = edge_h.reshape(B, TE, De)
    idx2 = edge_idx.reshape(B, TE, 1)
    mi2 = mask_i.reshape(B, N, 1)
    mij2 = mask_ij.reshape(B, TE, 1)

    def btile(rows_, feat):
        return pl.BlockSpec((None, rows_, feat), lambda b: (b, 0, 0))

    def rep(shape):
        return pl.BlockSpec(shape, lambda b: (0,) * len(shape))

    def rowblk(shape, i):
        return pl.BlockSpec(shape, lambda b, i=i: (i, 0))

    in_specs = [
        btile(N, Dn),
        btile(TE, De),
        btile(TE, 1),
        btile(N, 1),
        btile(TE, 1),
        rep((1, Dn)), rep((1, Dn)),
        rep((De, 1)), rep((De, 1)),
        rowblk((Dn, Hm), 0),
        rowblk((Dn, Hm), 1),
        rowblk((De, Hm), 2),
        rep((1, Hm)), rep((Hm, Dn)), rep((1, Dn)),
        rowblk((Dn, Hu), 0),
        rowblk((Dn, Hu), 1),
        rep((1, Hu)), rep((Hu, Dn)), rep((1, Dn)),
        rowblk((Dn, He), 0),
        rowblk((Dn, He), 1),
        rowblk((De, He), 2),
        rep((1, He)), rep((He, De)), rep((1, De)),
    ]
    out_specs = (btile(N, Dn), btile(TE, De))
    out_shape = (jax.ShapeDtypeStruct((B, N, Dn), f32),
                 jax.ShapeDtypeStruct((B, TE, De), f32))

    node_out, edge_out = pl.pallas_call(
        functools.partial(_fused_kernel, K=K, C=C),
        out_shape=out_shape,
        grid=(B,),
        in_specs=in_specs,
        out_specs=out_specs,
        scratch_shapes=[pltpu.VMEM((TE, De), jnp.bfloat16)],
        compiler_params=pltpu.CompilerParams(
            dimension_semantics=("parallel",),
            vmem_limit_bytes=VMEM_LIMIT),
    )(node_h, eh2, idx2, mi2, mij2,
      node_norm_w.reshape(1, Dn), node_norm_b.reshape(1, Dn),
      edge_norm_w.reshape(De, 1), edge_norm_b.reshape(De, 1),
      msg_W1, msg_W1, msg_W1,
      msg_b1.reshape(1, Hm), msg_W2, msg_b2.reshape(1, Dn),
      upd_W1, upd_W1,
      upd_b1.reshape(1, Hu), upd_W2, upd_b2.reshape(1, Dn),
      edge_W1, edge_W1, edge_W1,
      edge_b1.reshape(1, He), edge_W2, edge_b2.reshape(1, De))

    return node_out, edge_out.reshape(B, N, K, De)


# speedup vs baseline: 1.0062x; 1.0062x over previous
import functools

import jax
import jax.numpy as jnp
from jax import lax
from jax.experimental import pallas as pl
from jax.experimental.pallas import tpu as pltpu

EPS = 1e-5
VMEM_LIMIT = 64 * 1024 * 1024


def _fused_kernel(nh_ref, eh_ref, idx_ref, mi_ref, mij_ref, mijnk_ref,
                  nnw_ref, nnb_ref, enw_ref, enbc_ref,
                  wmi_ref, wmj_ref, wme_ref, mb1_ref, mw2_ref, mb2_ref,
                  u1n_ref, u1m_ref, ub1_ref, uw2_ref, ub2_ref,
                  wei_ref, wej_ref, wee_ref, eb1_ref, ew2_ref, eb2_ref,
                  no_ref, eo_ref,
                  oh_scr, en_scr,
                  *, K, C):
    N, Dn = nh_ref.shape
    TE, De = eh_ref.shape
    CH = TE // C
    Tc = CH // K
    bf16 = jnp.bfloat16
    f32 = jnp.float32

    enw = enw_ref[...]
    enbc = enbc_ref[...]
    wme = (enw * wme_ref[...]).astype(bf16)
    wee = (enw * wee_ref[...]).astype(bf16)
    mb1 = mb1_ref[...] + jnp.sum(enbc * wme_ref[...], axis=0, keepdims=True)
    eb1 = eb1_ref[...] + jnp.sum(enbc * wee_ref[...], axis=0, keepdims=True)
    wmi = wmi_ref[...].astype(bf16)
    wmj = wmj_ref[...].astype(bf16)
    wei = wei_ref[...].astype(bf16)
    wej = wej_ref[...].astype(bf16)
    u1n = u1n_ref[...].astype(bf16)
    u1m = u1m_ref[...].astype(bf16)
    mw2 = mw2_ref[...].astype(bf16)
    uw2 = uw2_ref[...].astype(bf16)
    ew2 = ew2_ref[...].astype(bf16)

    s_sel = (lax.broadcasted_iota(jnp.int32, (Tc, CH), 1) // K
             == lax.broadcasted_iota(jnp.int32, (Tc, CH), 0)).astype(bf16)

    nh = nh_ref[...]
    mi = mi_ref[...]
    mu = jnp.mean(nh, axis=-1, keepdims=True)
    var = jnp.mean(nh * nh, axis=-1, keepdims=True) - mu * mu
    nhn = ((nh - mu) * lax.rsqrt(var + EPS) * nnw_ref[...] + nnb_ref[...]) * mi
    nhn_b = nhn.astype(bf16)

    pre_i = jnp.dot(nhn_b, wmi, preferred_element_type=f32) + mb1
    tblw_m = jnp.dot(nhn_b, wmj, preferred_element_type=f32).astype(bf16)

    iota = lax.broadcasted_iota(jnp.int32, (CH, N), 1)

    hsums = []
    for c in range(C):
        sl = pl.ds(c * CH, CH)
        e = eh_ref[sl, :]
        mu_e = jnp.mean(e, axis=-1, keepdims=True)
        var_e = jnp.mean(e * e, axis=-1, keepdims=True) - mu_e * mu_e
        mij = mij_ref[sl, :]
        en_b = ((e - mu_e) * lax.rsqrt(var_e + EPS)).astype(bf16)
        en_scr[sl, :] = en_b

        oh = (idx_ref[sl, :] == iota).astype(bf16)
        oh_scr[sl, :] = oh

        zj = jnp.dot(oh, tblw_m, preferred_element_type=f32)
        ze = jnp.dot(en_b, wme, preferred_element_type=f32)
        z = (zj + ze).reshape(Tc, K, -1) + pre_i[c * Tc:(c + 1) * Tc][:, None, :]
        h = (jnp.maximum(z, 0.0).reshape(CH, -1) * mij).astype(bf16)
        hsums.append(jnp.dot(s_sel, h, preferred_element_type=f32))

    hsum = jnp.concatenate(hsums, axis=0)
    msum = jnp.sum(mijnk_ref[...], axis=-1, keepdims=True)
    msg = (jnp.dot(hsum.astype(bf16), mw2, preferred_element_type=f32)
           + mb2_ref[...] * msum)

    u = jnp.maximum(
        jnp.dot(nhn_b, u1n, preferred_element_type=f32)
        + jnp.dot(msg.astype(bf16), u1m, preferred_element_type=f32)
        + ub1_ref[...], 0.0)
    upd = jnp.dot(u.astype(bf16), uw2, preferred_element_type=f32) + ub2_ref[...]
    nout = (nh + upd) * mi
    no_ref[...] = nout

    nout_b = nout.astype(bf16)
    pre_e = jnp.dot(nout_b, wei, preferred_element_type=f32) + eb1
    tblw_e = jnp.dot(nout_b, wej, preferred_element_type=f32).astype(bf16)

    for c in range(C):
        sl = pl.ds(c * CH, CH)
        oh = oh_scr[sl, :]
        en_b = en_scr[sl, :]
        zj = jnp.dot(oh, tblw_e, preferred_element_type=f32)
        ze = jnp.dot(en_b, wee, preferred_element_type=f32)
        z = (zj + ze).reshape(Tc, K, -1) + pre_e[c * Tc:(c + 1) * Tc][:, None, :]
        h = jnp.maximum(z, 0.0).reshape(CH, -1)
        upd_e = jnp.dot(h.astype(bf16), ew2, preferred_element_type=f32) + eb2_ref[...]
        eo_ref[sl, :] = (eh_ref[sl, :] + upd_e) * mij_ref[sl, :]


def kernel(node_h, edge_h, edge_idx, mask_i, mask_ij,
           node_norm_w, node_norm_b, edge_norm_w, edge_norm_b,
           msg_W1, msg_b1, msg_W2, msg_b2,
           upd_W1, upd_b1, upd_W2, upd_b2,
           edge_W1, edge_b1, edge_W2, edge_b2):
    B, N, Dn = node_h.shape
    K = edge_idx.shape[-1]
    De = edge_h.shape[-1]
    Hm = msg_W2.shape[0]
    Hu = upd_W2.shape[0]
    He = edge_W2.shape[0]
    TE = N * K
    C = 8
    f32 = jnp.float32

    eh2 = edge_h.reshape(B, TE, De)
    idx2 = edge_idx.reshape(B, TE, 1)
    mi2 = mask_i.reshape(B, N, 1)
    mij2 = mask_ij.reshape(B, TE, 1)

    def btile(rows_, feat):
        return pl.BlockSpec((None, rows_, feat), lambda b: (b, 0, 0))

    def rep(shape):
        return pl.BlockSpec(shape, lambda b: (0,) * len(shape))

    def rowblk(shape, i):
        return pl.BlockSpec(shape, lambda b, i=i: (i, 0))

    in_specs = [
        btile(N, Dn),
        btile(TE, De),
        btile(TE, 1),
        btile(N, 1),
        btile(TE, 1),
        btile(N, K),
        rep((1, Dn)), rep((1, Dn)),
        rep((De, 1)), rep((De, 1)),
        rowblk((Dn, Hm), 0),
        rowblk((Dn, Hm), 1),
        rowblk((De, Hm), 2),
        rep((1, Hm)), rep((Hm, Dn)), rep((1, Dn)),
        rowblk((Dn, Hu), 0),
        rowblk((Dn, Hu), 1),
        rep((1, Hu)), rep((Hu, Dn)), rep((1, Dn)),
        rowblk((Dn, He), 0),
        rowblk((Dn, He), 1),
        rowblk((De, He), 2),
        rep((1, He)), rep((He, De)), rep((1, De)),
    ]
    out_specs = (btile(N, Dn), btile(TE, De))
    out_shape = (jax.ShapeDtypeStruct((B, N, Dn), f32),
                 jax.ShapeDtypeStruct((B, TE, De), f32))

    node_out, edge_out = pl.pallas_call(
        functools.partial(_fused_kernel, K=K, C=C),
        out_shape=out_shape,
        grid=(B,),
        in_specs=in_specs,
        out_specs=out_specs,
        scratch_shapes=[pltpu.VMEM((TE, N), jnp.bfloat16),
                        pltpu.VMEM((TE, De), jnp.bfloat16)],
        compiler_params=pltpu.CompilerParams(
            dimension_semantics=("parallel",),
            vmem_limit_bytes=VMEM_LIMIT),
    )(node_h, eh2, idx2, mi2, mij2, mask_ij,
      node_norm_w.reshape(1, Dn), node_norm_b.reshape(1, Dn),
      edge_norm_w.reshape(De, 1), edge_norm_b.reshape(De, 1),
      msg_W1, msg_W1, msg_W1,
      msg_b1.reshape(1, Hm), msg_W2, msg_b2.reshape(1, Dn),
      upd_W1, upd_W1,
      upd_b1.reshape(1, Hu), upd_W2, upd_b2.reshape(1, Dn),
      edge_W1, edge_W1, edge_W1,
      edge_b1.reshape(1, He), edge_W2, edge_b2.reshape(1, De))

    return node_out, edge_out.reshape(B, N, K, De)
